# baseline (speedup 1.0000x reference)
import functools

import numpy as np
import jax
import jax.numpy as jnp
from jax import lax
from jax.experimental import pallas as pl
from jax.experimental.pallas import tpu as pltpu

K_NN = 20
_RSQ = np.float32(1.0) / np.sqrt(np.float32(1.0 + 1e-5))


def _pd_body(x_ref, xc_ref, o_ref):
    n = x_ref.shape[1]
    c = x_ref.shape[2]
    inner = -2.0 * jnp.dot(x_ref[0], xc_ref[0],
                           preferred_element_type=jnp.float32)
    if c == 3:
        xq = xc_ref[0] ** 2
        xx = (xq[0:1, :] + xq[1:2, :]) + xq[2:3, :]
    else:
        xx = jnp.sum(xc_ref[0] ** 2, axis=0, keepdims=True)
    o_ref[0] = (-xx) - inner - xx.reshape(n, 1)


def _pairwise_pd(x):
    b, n, c = x.shape
    xc = jnp.transpose(x, (0, 2, 1))
    return pl.pallas_call(
        _pd_body,
        grid=(b,),
        in_specs=[pl.BlockSpec((1, n, c), lambda i: (i, 0, 0)),
                  pl.BlockSpec((1, c, n), lambda i: (i, 0, 0))],
        out_specs=pl.BlockSpec((1, n, n), lambda i: (i, 0, 0)),
        out_shape=jax.ShapeDtypeStruct((b, n, n), jnp.float32),
    )(x, xc)


def _edge_body(g_ref, xn_ref, wt_ref, gg_ref, bb_ref, o_ref):
    nb, k, c = g_ref.shape
    co = wt_ref.shape[1]
    xn = xn_ref[...]
    d = g_ref[...] - xn[:, None, :]
    xb = jnp.broadcast_to(xn[:, None, :], (nb, k, c))
    e = jnp.concatenate((d, xb), axis=2).reshape(nb * k, 2 * c)
    h = jnp.dot(e, wt_ref[...], preferred_element_type=jnp.float32)
    h = h * gg_ref[...] + bb_ref[...]
    h = jnp.where(h >= 0, h, 0.2 * h)
    o_ref[...] = jnp.max(h.reshape(nb, k, co), axis=1)


def _edge_layer_exact(gathered, x, w, g, b):
    bsz, n, k, c = gathered.shape
    co = w.shape[0]
    gf = gathered.reshape(bsz * n, k, c)
    xf = x.reshape(bsz * n, c)
    nb = 256
    out = pl.pallas_call(
        _edge_body,
        grid=(bsz * n // nb,),
        in_specs=[
            pl.BlockSpec((nb, k, c), lambda i: (i, 0, 0)),
            pl.BlockSpec((nb, c), lambda i: (i, 0)),
            pl.BlockSpec((2 * c, co), lambda i: (0, 0)),
            pl.BlockSpec((1, co), lambda i: (0, 0)),
            pl.BlockSpec((1, co), lambda i: (0, 0)),
        ],
        out_specs=pl.BlockSpec((nb, co), lambda i: (i, 0)),
        out_shape=jax.ShapeDtypeStruct((bsz * n, co), jnp.float32),
    )(gf, xf, w.T, (g * _RSQ).reshape(1, co), b.reshape(1, co))
    return out.reshape(bsz, n, co)


def _gather_rows(x, idx):
    return jax.vmap(lambda xb, ib: xb[ib])(x, idx)


def _dense_tail_kernel(h_ref, w4t_ref, gg_ref, bb_ref, out_ref):
    q = jnp.dot(h_ref[0], w4t_ref[...], preferred_element_type=jnp.float32)
    y = q * gg_ref[...] + bb_ref[...]
    y = jnp.where(y >= 0, y, 0.2 * y)
    m = jnp.max(y, axis=0, keepdims=True)
    m = jnp.broadcast_to(m, (8, 1024))

    @pl.when(pl.program_id(1) == 0)
    def _init():
        out_ref[0] = m

    @pl.when(pl.program_id(1) != 0)
    def _acc():
        out_ref[0] = jnp.maximum(out_ref[0], m)


def _mlp_kernel(hm_ref, l1w_ref, l1b_ref, gg5_ref, bb5_ref,
                l2w_ref, l2b_ref, gg6_ref, bb6_ref,
                l3w_ref, l3b_ref, out_ref):
    h = hm_ref[...]
    h = jnp.dot(h, l1w_ref[...], preferred_element_type=jnp.float32) + l1b_ref[...]
    h = h * gg5_ref[...] + bb5_ref[...]
    h = jnp.where(h >= 0, h, 0.2 * h)
    h = jnp.dot(h, l2w_ref[...], preferred_element_type=jnp.float32) + l2b_ref[...]
    h = h * gg6_ref[...] + bb6_ref[...]
    h = jnp.where(h >= 0, h, 0.2 * h)
    out_ref[...] = (
        jnp.dot(h, l3w_ref[...], preferred_element_type=jnp.float32) + l3b_ref[...]
    )


def kernel(x, w1, w2, w3, w4, lin1_w, lin1_b, lin2_w, lin2_b, lin3_w, lin3_b,
           g1, b1, g2, b2, g3, b3, g4, b4, g5, b5, g6, b6):
    B, N, _ = x.shape

    idx1 = lax.top_k(_pairwise_pd(x), K_NN)[1]
    x1 = _edge_layer_exact(_gather_rows(x, idx1), x, w1, g1, b1)

    idx2 = lax.top_k(_pairwise_pd(x1), K_NN)[1]
    x2 = _edge_layer_exact(_gather_rows(x1, idx2), x1, w2, g2, b2)

    idx3 = lax.top_k(_pairwise_pd(x2), K_NN)[1]
    x3 = _edge_layer_exact(_gather_rows(x2, idx3), x2, w3, g3, b3)

    h = jnp.concatenate((x1, x2, x3), axis=-1)

    NB = 512
    pooled = pl.pallas_call(
        _dense_tail_kernel,
        grid=(B, N // NB),
        in_specs=[
            pl.BlockSpec((1, NB, 256), lambda b_, n_: (b_, n_, 0)),
            pl.BlockSpec((256, 1024), lambda b_, n_: (0, 0)),
            pl.BlockSpec((1, 1024), lambda b_, n_: (0, 0)),
            pl.BlockSpec((1, 1024), lambda b_, n_: (0, 0)),
        ],
        out_specs=pl.BlockSpec((1, 8, 1024), lambda b_, n_: (b_, 0, 0)),
        out_shape=jax.ShapeDtypeStruct((B, 8, 1024), jnp.float32),
    )(h, w4.T, (g4 * _RSQ).reshape(1, 1024), b4.reshape(1, 1024))[:, 0, :]

    out = pl.pallas_call(
        _mlp_kernel,
        out_shape=jax.ShapeDtypeStruct((B, 2), jnp.float32),
    )(pooled, lin1_w.T, lin1_b.reshape(1, 512),
      (g5 * _RSQ).reshape(1, 512), b5.reshape(1, 512),
      lin2_w.T, lin2_b.reshape(1, 256),
      (g6 * _RSQ).reshape(1, 256), b6.reshape(1, 256),
      lin3_w.T, lin3_b.reshape(1, 2))
    return out


# speedup vs baseline: 4.8162x; 4.8162x over previous
import functools

import numpy as np
import jax
import jax.numpy as jnp
from jax import lax
from jax.experimental import pallas as pl
from jax.experimental.pallas import tpu as pltpu
from jax.experimental.pallas import tpu_sc as plsc

K_NN = 20
K_PAD = 24
_RSQ = np.float32(1.0) / np.sqrt(np.float32(1.0 + 1e-5))

_NC, _NS, _LANES = 2, 16, 16
_NW = _NC * _NS


def _sc_topk_gather_body(nrows, nfeat, pd_hbm, cm_hbm, src_hbm, out_hbm,
                         pda, pdb, pdc, pdd, cma, cmb, cmc, cmd,
                         idxa, idxb, idxc, idxd, rowsa, rowsb, rowsc, rowsd,
                         sem_pa, sem_pb, sem_pc, sem_pd,
                         sem_ca, sem_cb, sem_cc, sem_cd,
                         sem_ga, sem_gb, sem_gc, sem_gd):
    wid = lax.axis_index("s") * _NC + lax.axis_index("c")
    row0 = wid * nrows
    base = (row0 // 2048) * 2048
    iota = lax.broadcasted_iota(jnp.int32, (_LANES,), 0)
    lane0 = iota == 0
    neg_inf = jnp.full((_LANES,), -jnp.inf, dtype=jnp.float32)

    dnums = lax.GatherDimensionNumbers(offset_dims=(),
                                       collapsed_slice_dims=(0,),
                                       start_index_map=(0,))

    def perm(v, idx):
        return lax.gather(v, idx[:, None], dimension_numbers=dnums,
                          slice_sizes=(1,),
                          mode=lax.GatherScatterMode.PROMISE_IN_BOUNDS)

    def xmax(v):
        for sh in (8, 4, 2, 1):
            v = jnp.maximum(v, perm(v, iota ^ sh))
        return v

    def xmin(v):
        for sh in (8, 4, 2, 1):
            v = jnp.minimum(v, perm(v, iota ^ sh))
        return v

    pdrefs = (pda, pdb, pdc, pdd)
    cmrefs = (cma, cmb, cmc, cmd)
    idxrefs = (idxa, idxb, idxc, idxd)
    rowrefs = (rowsa, rowsb, rowsc, rowsd)
    sem_p = (sem_pa, sem_pb, sem_pc, sem_pd)
    sem_c = (sem_ca, sem_cb, sem_cc, sem_cd)
    sem_g = (sem_ga, sem_gb, sem_gc, sem_gd)

    def issue(r, ph):
        pltpu.async_copy(pd_hbm.at[r], pdrefs[ph], sem_p[ph])
        pltpu.async_copy(cm_hbm.at[r], cmrefs[ph], sem_c[ph])

    def wait(r, ph):
        pltpu.make_async_copy(pd_hbm.at[r], pdrefs[ph], sem_p[ph]).wait()
        pltpu.make_async_copy(cm_hbm.at[r], cmrefs[ph], sem_c[ph]).wait()

    def extract_step(it, ph, st):
        pdrow, cmref, idxref = pdrefs[ph], cmrefs[ph], idxrefs[ph]
        cms = [cmref[pl.ds(16 * j, 16)] for j in range(8)]
        m01 = jnp.maximum(cms[0], cms[1])
        m23 = jnp.maximum(cms[2], cms[3])
        m45 = jnp.maximum(cms[4], cms[5])
        m67 = jnp.maximum(cms[6], cms[7])
        m = jnp.maximum(jnp.maximum(m01, m23), jnp.maximum(m45, m67))
        sv = xmax(m)
        cand = jnp.full((_LANES,), 4096, dtype=jnp.int32)
        for j in range(8):
            cand = jnp.minimum(cand,
                               jnp.where(cms[j] == sv, iota + (16 * j), 4096))
        cbest = xmin(cand)
        ch_idx = cbest * 16 + iota
        ch = plsc.load_gather(pdrow, [ch_idx])
        lane = xmin(jnp.where(ch == sv, iota, 16))
        col = cbest * 16 + lane
        plsc.store_scatter(idxref, [jnp.full((_LANES,), it, jnp.int32)],
                           col + base, mask=lane0)
        chm = jnp.where(iota == lane, neg_inf, ch)
        plsc.store_scatter(pdrow, [col], neg_inf, mask=lane0)
        plsc.store_scatter(cmref, [cbest], xmax(chm), mask=lane0)
        return st

    selfsplat = jnp.full((_LANES,), base, jnp.int32)

    def quad_body(p2, st):
        for half in range(2):
            b0, b1 = 2 * half, 2 * half + 1
            r0 = row0 + 4 * p2 + 2 * half
            wait(r0, b0)
            wait(r0 + 1, b1)
            for ph in (b0, b1):
                idxr = idxrefs[ph]
                idxr[pl.ds(0, 16)] = selfsplat
                idxr[pl.ds(8, 16)] = selfsplat
            for it in range(K_NN):
                for ph in (b0, b1):
                    st = extract_step(it, ph, st)
            pltpu.async_copy(src_hbm.at[idxrefs[b0]], rowrefs[b0], sem_g[b0])
            pltpu.async_copy(src_hbm.at[idxrefs[b1]], rowrefs[b1], sem_g[b1])
            @pl.when(r0 + 4 < row0 + nrows)
            def _pf():
                issue(r0 + 4, b0)
                issue(r0 + 5, b1)
            pltpu.make_async_copy(src_hbm.at[idxrefs[b0]], rowrefs[b0],
                                  sem_g[b0]).wait()
            pltpu.sync_copy(rowrefs[b0], out_hbm.at[r0])
            pltpu.make_async_copy(src_hbm.at[idxrefs[b1]], rowrefs[b1],
                                  sem_g[b1]).wait()
            pltpu.sync_copy(rowrefs[b1], out_hbm.at[r0 + 1])
        return st

    issue(row0, 0)
    issue(row0 + 1, 1)
    issue(row0 + 2, 2)
    issue(row0 + 3, 3)
    lax.fori_loop(0, nrows // 4, quad_body, 0)


def _sc_topk_gather(pd, cm, src):
    bn, n = pd.shape
    c = src.shape[1]
    nrows = bn // _NW
    mesh = plsc.VectorSubcoreMesh(core_axis_name="c", subcore_axis_name="s",
                                  num_cores=_NC, num_subcores=_NS)
    body = functools.partial(_sc_topk_gather_body, nrows, c)
    return pl.kernel(
        body,
        out_type=jax.ShapeDtypeStruct((bn, K_PAD, c), jnp.float32),
        mesh=mesh,
        compiler_params=pltpu.CompilerParams(needs_layout_passes=False),
        scratch_types=(
            [pltpu.VMEM((n,), jnp.float32)] * 4
            + [pltpu.VMEM((128,), jnp.float32)] * 4
            + [pltpu.VMEM((K_PAD,), jnp.int32)] * 4
            + [pltpu.VMEM((K_PAD, c), jnp.float32)] * 4
            + [pltpu.SemaphoreType.DMA] * 12
        ),
    )(pd, cm, src)


def _pd_body(x_ref, xc_ref, o_ref):
    n = x_ref.shape[1]
    c = x_ref.shape[2]
    inner = -2.0 * jnp.dot(x_ref[0], xc_ref[0],
                           preferred_element_type=jnp.float32)
    if c == 3:
        xq = xc_ref[0] ** 2
        xx = (xq[0:1, :] + xq[1:2, :]) + xq[2:3, :]
    else:
        xx = jnp.sum(xc_ref[0] ** 2, axis=0, keepdims=True)
    o_ref[0] = (-xx) - inner - xx.reshape(n, 1)


def _pairwise_pd(x):
    b, n, c = x.shape
    xc = jnp.transpose(x, (0, 2, 1))
    return pl.pallas_call(
        _pd_body,
        grid=(b,),
        in_specs=[pl.BlockSpec((1, n, c), lambda i: (i, 0, 0)),
                  pl.BlockSpec((1, c, n), lambda i: (i, 0, 0))],
        out_specs=pl.BlockSpec((1, n, n), lambda i: (i, 0, 0)),
        out_shape=jax.ShapeDtypeStruct((b, n, n), jnp.float32),
    )(x, xc)


def _edge_body(g_ref, xn_ref, wt_ref, gg_ref, bb_ref, o_ref):
    nb, k, c = g_ref.shape
    co = wt_ref.shape[1]
    xn = xn_ref[...]
    d = g_ref[...] - xn[:, None, :]
    xb = jnp.broadcast_to(xn[:, None, :], (nb, k, c))
    e = jnp.concatenate((d, xb), axis=2).reshape(nb * k, 2 * c)
    h = jnp.dot(e, wt_ref[...], preferred_element_type=jnp.float32)
    h = h * gg_ref[...] + bb_ref[...]
    h = jnp.where(h >= 0, h, 0.2 * h)
    o_ref[...] = jnp.max(h.reshape(nb, k, co), axis=1)


def _edge_layer_exact(gathered, x, w, g, b):
    bsz, n, k, c = gathered.shape
    co = w.shape[0]
    gf = gathered.reshape(bsz * n, k, c)
    xf = x.reshape(bsz * n, c)
    nb = 256
    out = pl.pallas_call(
        _edge_body,
        grid=(bsz * n // nb,),
        in_specs=[
            pl.BlockSpec((nb, k, c), lambda i: (i, 0, 0)),
            pl.BlockSpec((nb, c), lambda i: (i, 0)),
            pl.BlockSpec((2 * c, co), lambda i: (0, 0)),
            pl.BlockSpec((1, co), lambda i: (0, 0)),
            pl.BlockSpec((1, co), lambda i: (0, 0)),
        ],
        out_specs=pl.BlockSpec((nb, co), lambda i: (i, 0)),
        out_shape=jax.ShapeDtypeStruct((bsz * n, co), jnp.float32),
    )(gf, xf, w.T, (g * _RSQ).reshape(1, co), b.reshape(1, co))
    return out.reshape(bsz, n, co)


def _gather_rows(x, idx):
    return jax.vmap(lambda xb, ib: xb[ib])(x, idx)


def _knn_gather_sc(xfeat):
    b, n, c = xfeat.shape
    pd = _pairwise_pd(xfeat)
    pdf = pd.reshape(b * n, n)
    cm = jnp.max(pd.reshape(b, n, n // 16, 16), axis=-1).reshape(b * n, n // 16)
    cpad = 128
    src = xfeat.reshape(b * n, c)
    if cpad != c:
        src = jnp.pad(src, ((0, 0), (0, cpad - c)))
    g = _sc_topk_gather(pdf, cm, src)
    return g[:, :K_NN, :c].reshape(b, n, K_NN, c)


def _dense_tail_kernel(h_ref, w4t_ref, gg_ref, bb_ref, out_ref):
    q = jnp.dot(h_ref[0], w4t_ref[...], preferred_element_type=jnp.float32)
    y = q * gg_ref[...] + bb_ref[...]
    y = jnp.where(y >= 0, y, 0.2 * y)
    m = jnp.max(y, axis=0, keepdims=True)
    m = jnp.broadcast_to(m, (8, 1024))

    @pl.when(pl.program_id(1) == 0)
    def _init():
        out_ref[0] = m

    @pl.when(pl.program_id(1) != 0)
    def _acc():
        out_ref[0] = jnp.maximum(out_ref[0], m)


def _mlp_kernel(hm_ref, l1w_ref, l1b_ref, gg5_ref, bb5_ref,
                l2w_ref, l2b_ref, gg6_ref, bb6_ref,
                l3w_ref, l3b_ref, out_ref):
    h = hm_ref[...]
    h = jnp.dot(h, l1w_ref[...], preferred_element_type=jnp.float32) + l1b_ref[...]
    h = h * gg5_ref[...] + bb5_ref[...]
    h = jnp.where(h >= 0, h, 0.2 * h)
    h = jnp.dot(h, l2w_ref[...], preferred_element_type=jnp.float32) + l2b_ref[...]
    h = h * gg6_ref[...] + bb6_ref[...]
    h = jnp.where(h >= 0, h, 0.2 * h)
    out_ref[...] = (
        jnp.dot(h, l3w_ref[...], preferred_element_type=jnp.float32) + l3b_ref[...]
    )


def kernel(x, w1, w2, w3, w4, lin1_w, lin1_b, lin2_w, lin2_b, lin3_w, lin3_b,
           g1, b1, g2, b2, g3, b3, g4, b4, g5, b5, g6, b6):
    B, N, _ = x.shape

    x1 = _edge_layer_exact(_knn_gather_sc(x), x, w1, g1, b1)

    x2 = _edge_layer_exact(_knn_gather_sc(x1), x1, w2, g2, b2)

    x3 = _edge_layer_exact(_knn_gather_sc(x2), x2, w3, g3, b3)

    h = jnp.concatenate((x1, x2, x3), axis=-1)

    NB = 512
    pooled = pl.pallas_call(
        _dense_tail_kernel,
        grid=(B, N // NB),
        in_specs=[
            pl.BlockSpec((1, NB, 256), lambda b_, n_: (b_, n_, 0)),
            pl.BlockSpec((256, 1024), lambda b_, n_: (0, 0)),
            pl.BlockSpec((1, 1024), lambda b_, n_: (0, 0)),
            pl.BlockSpec((1, 1024), lambda b_, n_: (0, 0)),
        ],
        out_specs=pl.BlockSpec((1, 8, 1024), lambda b_, n_: (b_, 0, 0)),
        out_shape=jax.ShapeDtypeStruct((B, 8, 1024), jnp.float32),
    )(h, w4.T, (g4 * _RSQ).reshape(1, 1024), b4.reshape(1, 1024))[:, 0, :]

    out = pl.pallas_call(
        _mlp_kernel,
        out_shape=jax.ShapeDtypeStruct((B, 2), jnp.float32),
    )(pooled, lin1_w.T, lin1_b.reshape(1, 512),
      (g5 * _RSQ).reshape(1, 512), b5.reshape(1, 512),
      lin2_w.T, lin2_b.reshape(1, 256),
      (g6 * _RSQ).reshape(1, 256), b6.reshape(1, 256),
      lin3_w.T, lin3_b.reshape(1, 2))
    return out


# speedup vs baseline: 6.8361x; 1.4194x over previous
import functools

import numpy as np
import jax
import jax.numpy as jnp
from jax import lax
from jax.experimental import pallas as pl
from jax.experimental.pallas import tpu as pltpu
from jax.experimental.pallas import tpu_sc as plsc

K_NN = 20
K_PAD = 24
_RSQ = np.float32(1.0) / np.sqrt(np.float32(1.0 + 1e-5))

_NC, _NS, _LANES = 2, 16, 16
_NW = _NC * _NS


def _sc_topk_gather_body(nrows, nfeat, pd_hbm, cm_hbm, src_hbm, out_hbm,
                         pda, pdb, pdc, pdd, cma, cmb, cmc, cmd,
                         idxa, idxb, idxc, idxd, rowsa, rowsb, rowsc, rowsd,
                         sem_pa, sem_pb, sem_pc, sem_pd,
                         sem_ca, sem_cb, sem_cc, sem_cd,
                         sem_ga, sem_gb, sem_gc, sem_gd):
    wid = lax.axis_index("s") * _NC + lax.axis_index("c")
    row0 = wid * nrows
    base = (row0 // 2048) * 2048
    iota = lax.broadcasted_iota(jnp.int32, (_LANES,), 0)
    lane0 = iota == 0
    neg_inf = jnp.full((_LANES,), -jnp.inf, dtype=jnp.float32)

    dnums = lax.GatherDimensionNumbers(offset_dims=(),
                                       collapsed_slice_dims=(0,),
                                       start_index_map=(0,))

    def perm(v, idx):
        return lax.gather(v, idx[:, None], dimension_numbers=dnums,
                          slice_sizes=(1,),
                          mode=lax.GatherScatterMode.PROMISE_IN_BOUNDS)

    def xmax(v):
        for sh in (8, 4, 2, 1):
            v = jnp.maximum(v, perm(v, iota ^ sh))
        return v

    def xmin(v):
        for sh in (8, 4, 2, 1):
            v = jnp.minimum(v, perm(v, iota ^ sh))
        return v

    pdrefs = (pda, pdb, pdc, pdd)
    cmrefs = (cma, cmb, cmc, cmd)
    idxrefs = (idxa, idxb, idxc, idxd)
    rowrefs = (rowsa, rowsb, rowsc, rowsd)
    sem_p = (sem_pa, sem_pb, sem_pc, sem_pd)
    sem_c = (sem_ca, sem_cb, sem_cc, sem_cd)
    sem_g = (sem_ga, sem_gb, sem_gc, sem_gd)

    def issue(r, ph):
        pltpu.async_copy(pd_hbm.at[r], pdrefs[ph], sem_p[ph])
        pltpu.async_copy(cm_hbm.at[r], cmrefs[ph], sem_c[ph])

    def wait(r, ph):
        pltpu.make_async_copy(pd_hbm.at[r], pdrefs[ph], sem_p[ph]).wait()
        pltpu.make_async_copy(cm_hbm.at[r], cmrefs[ph], sem_c[ph]).wait()

    def extract_step(it, ph, cms):
        pdrow, idxref = pdrefs[ph], idxrefs[ph]
        m01 = jnp.maximum(cms[0], cms[1])
        m23 = jnp.maximum(cms[2], cms[3])
        m45 = jnp.maximum(cms[4], cms[5])
        m67 = jnp.maximum(cms[6], cms[7])
        m = jnp.maximum(jnp.maximum(m01, m23), jnp.maximum(m45, m67))
        sv = xmax(m)
        cand = jnp.full((_LANES,), 4096, dtype=jnp.int32)
        for j in range(8):
            cand = jnp.minimum(cand,
                               jnp.where(cms[j] == sv, iota + (16 * j), 4096))
        cbest = xmin(cand)
        ch_idx = cbest * 16 + iota
        ch = plsc.load_gather(pdrow, [ch_idx])
        lane = xmin(jnp.where(ch == sv, iota, 16))
        col = cbest * 16 + lane
        plsc.store_scatter(idxref, [jnp.full((_LANES,), it, jnp.int32)],
                           col + base, mask=lane0)
        chm = jnp.where(iota == lane, neg_inf, ch)
        plsc.store_scatter(pdrow, [col], neg_inf, mask=lane0)
        nm = xmax(chm)
        jsel = cbest >> 4
        lsel = cbest & 15
        return [jnp.where((jsel == j) & (iota == lsel), nm, cms[j])
                for j in range(8)]

    selfsplat = jnp.full((_LANES,), base, jnp.int32)

    def quad_body(p2, st):
        for half in range(2):
            b0, b1 = 2 * half, 2 * half + 1
            r0 = row0 + 4 * p2 + 2 * half
            wait(r0, b0)
            wait(r0 + 1, b1)
            for ph in (b0, b1):
                idxr = idxrefs[ph]
                idxr[pl.ds(0, 16)] = selfsplat
                idxr[pl.ds(8, 16)] = selfsplat
            cms0 = [cmrefs[b0][pl.ds(16 * j, 16)] for j in range(8)]
            cms1 = [cmrefs[b1][pl.ds(16 * j, 16)] for j in range(8)]
            for it in range(K_NN):
                cms0 = extract_step(it, b0, cms0)
                cms1 = extract_step(it, b1, cms1)
            pltpu.async_copy(src_hbm.at[idxrefs[b0]], rowrefs[b0], sem_g[b0])
            pltpu.async_copy(src_hbm.at[idxrefs[b1]], rowrefs[b1], sem_g[b1])
            @pl.when(r0 + 4 < row0 + nrows)
            def _pf():
                issue(r0 + 4, b0)
                issue(r0 + 5, b1)
            pltpu.make_async_copy(src_hbm.at[idxrefs[b0]], rowrefs[b0],
                                  sem_g[b0]).wait()
            pltpu.sync_copy(rowrefs[b0], out_hbm.at[r0])
            pltpu.make_async_copy(src_hbm.at[idxrefs[b1]], rowrefs[b1],
                                  sem_g[b1]).wait()
            pltpu.sync_copy(rowrefs[b1], out_hbm.at[r0 + 1])
        return st

    issue(row0, 0)
    issue(row0 + 1, 1)
    issue(row0 + 2, 2)
    issue(row0 + 3, 3)
    lax.fori_loop(0, nrows // 4, quad_body, 0)


def _sc_topk_gather(pd, cm, src):
    bn, n = pd.shape
    c = src.shape[1]
    nrows = bn // _NW
    mesh = plsc.VectorSubcoreMesh(core_axis_name="c", subcore_axis_name="s",
                                  num_cores=_NC, num_subcores=_NS)
    body = functools.partial(_sc_topk_gather_body, nrows, c)
    return pl.kernel(
        body,
        out_type=jax.ShapeDtypeStruct((bn, K_PAD, c), jnp.float32),
        mesh=mesh,
        compiler_params=pltpu.CompilerParams(needs_layout_passes=False),
        scratch_types=(
            [pltpu.VMEM((n,), jnp.float32)] * 4
            + [pltpu.VMEM((128,), jnp.float32)] * 4
            + [pltpu.VMEM((K_PAD,), jnp.int32)] * 4
            + [pltpu.VMEM((K_PAD, c), jnp.float32)] * 4
            + [pltpu.SemaphoreType.DMA] * 12
        ),
    )(pd, cm, src)


def _pd_body(x_ref, xc_ref, o_ref):
    n = x_ref.shape[1]
    c = x_ref.shape[2]
    inner = -2.0 * jnp.dot(x_ref[0], xc_ref[0],
                           preferred_element_type=jnp.float32)
    if c == 3:
        xq = xc_ref[0] ** 2
        xx = (xq[0:1, :] + xq[1:2, :]) + xq[2:3, :]
    else:
        xx = jnp.sum(xc_ref[0] ** 2, axis=0, keepdims=True)
    o_ref[0] = (-xx) - inner - xx.reshape(n, 1)


def _pairwise_pd(x):
    b, n, c = x.shape
    xc = jnp.transpose(x, (0, 2, 1))
    return pl.pallas_call(
        _pd_body,
        grid=(b,),
        in_specs=[pl.BlockSpec((1, n, c), lambda i: (i, 0, 0)),
                  pl.BlockSpec((1, c, n), lambda i: (i, 0, 0))],
        out_specs=pl.BlockSpec((1, n, n), lambda i: (i, 0, 0)),
        out_shape=jax.ShapeDtypeStruct((b, n, n), jnp.float32),
    )(x, xc)


def _edge_body(g_ref, xn_ref, wt_ref, gg_ref, bb_ref, o_ref):
    nb, k, c = g_ref.shape
    co = wt_ref.shape[1]
    xn = xn_ref[...]
    d = g_ref[...] - xn[:, None, :]
    xb = jnp.broadcast_to(xn[:, None, :], (nb, k, c))
    e = jnp.concatenate((d, xb), axis=2).reshape(nb * k, 2 * c)
    h = jnp.dot(e, wt_ref[...], preferred_element_type=jnp.float32)
    h = h * gg_ref[...] + bb_ref[...]
    h = jnp.where(h >= 0, h, 0.2 * h)
    o_ref[...] = jnp.max(h.reshape(nb, k, co), axis=1)


def _edge_layer_exact(gathered, x, w, g, b):
    bsz, n, k, c = gathered.shape
    co = w.shape[0]
    gf = gathered.reshape(bsz * n, k, c)
    xf = x.reshape(bsz * n, c)
    nb = 256
    out = pl.pallas_call(
        _edge_body,
        grid=(bsz * n // nb,),
        in_specs=[
            pl.BlockSpec((nb, k, c), lambda i: (i, 0, 0)),
            pl.BlockSpec((nb, c), lambda i: (i, 0)),
            pl.BlockSpec((2 * c, co), lambda i: (0, 0)),
            pl.BlockSpec((1, co), lambda i: (0, 0)),
            pl.BlockSpec((1, co), lambda i: (0, 0)),
        ],
        out_specs=pl.BlockSpec((nb, co), lambda i: (i, 0)),
        out_shape=jax.ShapeDtypeStruct((bsz * n, co), jnp.float32),
    )(gf, xf, w.T, (g * _RSQ).reshape(1, co), b.reshape(1, co))
    return out.reshape(bsz, n, co)


def _gather_rows(x, idx):
    return jax.vmap(lambda xb, ib: xb[ib])(x, idx)


def _knn_gather_sc(xfeat):
    b, n, c = xfeat.shape
    pd = _pairwise_pd(xfeat)
    pdf = pd.reshape(b * n, n)
    cm = jnp.max(pd.reshape(b, n, n // 16, 16), axis=-1).reshape(b * n, n // 16)
    cpad = 128
    src = xfeat.reshape(b * n, c)
    if cpad != c:
        src = jnp.pad(src, ((0, 0), (0, cpad - c)))
    g = _sc_topk_gather(pdf, cm, src)
    return g[:, :K_NN, :c].reshape(b, n, K_NN, c)


def _dense_tail_kernel(h_ref, w4t_ref, gg_ref, bb_ref, out_ref):
    q = jnp.dot(h_ref[0], w4t_ref[...], preferred_element_type=jnp.float32)
    y = q * gg_ref[...] + bb_ref[...]
    y = jnp.where(y >= 0, y, 0.2 * y)
    m = jnp.max(y, axis=0, keepdims=True)
    m = jnp.broadcast_to(m, (8, 1024))

    @pl.when(pl.program_id(1) == 0)
    def _init():
        out_ref[0] = m

    @pl.when(pl.program_id(1) != 0)
    def _acc():
        out_ref[0] = jnp.maximum(out_ref[0], m)


def _mlp_kernel(hm_ref, l1w_ref, l1b_ref, gg5_ref, bb5_ref,
                l2w_ref, l2b_ref, gg6_ref, bb6_ref,
                l3w_ref, l3b_ref, out_ref):
    h = hm_ref[...]
    h = jnp.dot(h, l1w_ref[...], preferred_element_type=jnp.float32) + l1b_ref[...]
    h = h * gg5_ref[...] + bb5_ref[...]
    h = jnp.where(h >= 0, h, 0.2 * h)
    h = jnp.dot(h, l2w_ref[...], preferred_element_type=jnp.float32) + l2b_ref[...]
    h = h * gg6_ref[...] + bb6_ref[...]
    h = jnp.where(h >= 0, h, 0.2 * h)
    out_ref[...] = (
        jnp.dot(h, l3w_ref[...], preferred_element_type=jnp.float32) + l3b_ref[...]
    )


def kernel(x, w1, w2, w3, w4, lin1_w, lin1_b, lin2_w, lin2_b, lin3_w, lin3_b,
           g1, b1, g2, b2, g3, b3, g4, b4, g5, b5, g6, b6):
    B, N, _ = x.shape

    x1 = _edge_layer_exact(_knn_gather_sc(x), x, w1, g1, b1)

    x2 = _edge_layer_exact(_knn_gather_sc(x1), x1, w2, g2, b2)

    x3 = _edge_layer_exact(_knn_gather_sc(x2), x2, w3, g3, b3)

    h = jnp.concatenate((x1, x2, x3), axis=-1)

    NB = 512
    pooled = pl.pallas_call(
        _dense_tail_kernel,
        grid=(B, N // NB),
        in_specs=[
            pl.BlockSpec((1, NB, 256), lambda b_, n_: (b_, n_, 0)),
            pl.BlockSpec((256, 1024), lambda b_, n_: (0, 0)),
            pl.BlockSpec((1, 1024), lambda b_, n_: (0, 0)),
            pl.BlockSpec((1, 1024), lambda b_, n_: (0, 0)),
        ],
        out_specs=pl.BlockSpec((1, 8, 1024), lambda b_, n_: (b_, 0, 0)),
        out_shape=jax.ShapeDtypeStruct((B, 8, 1024), jnp.float32),
    )(h, w4.T, (g4 * _RSQ).reshape(1, 1024), b4.reshape(1, 1024))[:, 0, :]

    out = pl.pallas_call(
        _mlp_kernel,
        out_shape=jax.ShapeDtypeStruct((B, 2), jnp.float32),
    )(pooled, lin1_w.T, lin1_b.reshape(1, 512),
      (g5 * _RSQ).reshape(1, 512), b5.reshape(1, 512),
      lin2_w.T, lin2_b.reshape(1, 256),
      (g6 * _RSQ).reshape(1, 256), b6.reshape(1, 256),
      lin3_w.T, lin3_b.reshape(1, 2))
    return out
